# Initial kernel scaffold; baseline (speedup 1.0000x reference)
import jax
import jax.numpy as jnp
from jax.experimental import pallas as pl


def kernel(input, coord_feat, h0, lamda, alpha, l, edge_index, efeats, W_emb_in, b_emb_in, We1, be1, We2, be2, Wn1, bn1, Wn2, bn2, Wc1, bc1, Wc2, Wa, ba, W_emb_out, b_emb_out):
    raise NotImplementedError("write your pallas kernel here")


# speedup vs baseline: 2.5309x; 2.5309x over previous
import functools

import jax
import jax.numpy as jnp
from jax import lax
from jax.experimental import pallas as pl
from jax.experimental.pallas import tpu as pltpu
from jax.experimental.pallas import tpu_sc as plsc

N = 10000
NP = 10240
E = 160000
IN_SIZE = 256
HID = 512
EDGE_NF = 16
GW = HID + 128

NC = 2
NS = 16
NW = NC * NS

F32 = jnp.float32


def _mm(x, w):
    return lax.dot_general(x, w, (((1,), (1,)), ((), ())),
                           preferred_element_type=F32)


def _tc_precompute(inp, coordp, w_emb, b_emb, we1r, we1c, wn1h):
    TN = 2000

    def body(x_ref, cp_ref, wemb_ref, bemb_ref, wr_ref, wc_ref, wn_ref,
             hs_ref, hc_ref, hn_ref):
        h = _mm(x_ref[...], wemb_ref[...]) + bemb_ref[...]
        z = jnp.zeros((h.shape[0], GW - HID - 16), F32)
        cp = cp_ref[...]
        hs_ref[...] = jnp.concatenate([_mm(h, wr_ref[...]), cp, z], axis=1)
        hc_ref[...] = jnp.concatenate([_mm(h, wc_ref[...]), cp, z], axis=1)
        hn_ref[...] = _mm(h, wn_ref[...])

    full = lambda shape: pl.BlockSpec(shape, lambda i: (0,) * len(shape))
    return pl.pallas_call(
        body,
        grid=(N // TN,),
        in_specs=[
            pl.BlockSpec((TN, IN_SIZE), lambda i: (i, 0)),
            pl.BlockSpec((TN, 16), lambda i: (i, 0)),
            full((HID, IN_SIZE)),
            full((1, HID)),
            full((HID, HID)),
            full((HID, HID)),
            full((HID, HID)),
        ],
        out_specs=[
            pl.BlockSpec((TN, GW), lambda i: (i, 0)),
            pl.BlockSpec((TN, GW), lambda i: (i, 0)),
            pl.BlockSpec((TN, HID), lambda i: (i, 0)),
        ],
        out_shape=[
            jax.ShapeDtypeStruct((N, GW), F32),
            jax.ShapeDtypeStruct((N, GW), F32),
            jax.ShapeDtypeStruct((N, HID), F32),
        ],
    )(inp, coordp, w_emb, b_emb, we1r, we1c, wn1h)


def _sc_gather(hse, hce, row, col):
    EPW = E // NW
    B = 40
    NB = EPW // B
    mesh = plsc.VectorSubcoreMesh(core_axis_name="c", subcore_axis_name="s")

    @functools.partial(
        pl.kernel,
        mesh=mesh,
        out_type=(
            jax.ShapeDtypeStruct((E, GW), F32),
            jax.ShapeDtypeStruct((E, GW), F32),
        ),
        scratch_types=[
            pltpu.VMEM((B,), jnp.int32),
            pltpu.VMEM((B,), jnp.int32),
            pltpu.VMEM((B, GW), F32),
            pltpu.VMEM((B, GW), F32),
            pltpu.SemaphoreType.DMA,
            pltpu.SemaphoreType.DMA,
        ],
    )
    def k(hs_h, hc_h, row_h, col_h, g1_h, g2_h, rowv, colv, b1, b2, s1, s2):
        wid = lax.axis_index("s") * NC + lax.axis_index("c")

        @pl.loop(0, NB)
        def _(b):
            base = wid * EPW + b * B
            pltpu.sync_copy(row_h.at[pl.ds(base, B)], rowv)
            pltpu.sync_copy(col_h.at[pl.ds(base, B)], colv)
            cp1 = pltpu.async_copy(hs_h.at[rowv], b1, s1)
            cp2 = pltpu.async_copy(hc_h.at[colv], b2, s2)
            cp1.wait()
            cp2.wait()
            pltpu.sync_copy(b1, g1_h.at[pl.ds(base, B)])
            pltpu.sync_copy(b2, g2_h.at[pl.ds(base, B)])

    return k(hse, hce, row, col)


def _tc_edge(g1, g2, efeats, wrad8, we1ef, be1, we2, be2,
             wa_rep, ba128, wc1, bc1, wc2_rep):
    TE = 1280

    def body(g1h_ref, g1c_ref, g2h_ref, g2c_ref, ef_ref,
             wrad8_ref, we1ef_ref, be1_ref, we2_ref, be2_ref,
             wa_ref, ba_ref, wc1_ref, bc1_ref, wc2_ref,
             m5_ref):
        d = g1c_ref[...] - g2c_ref[...]
        d8 = d[:, 0:8]
        pre = (g1h_ref[...] + g2h_ref[...] + _mm(d8 * d8, wrad8_ref[...])
               + _mm(ef_ref[...], we1ef_ref[...]) + be1_ref[...])
        m1 = jax.nn.silu(pre)
        m2 = jax.nn.silu(_mm(m1, we2_ref[...]) + be2_ref[...])
        att = jax.nn.sigmoid(_mm(m2, wa_ref[...]) + ba_ref[...])
        attc = jnp.concatenate([att] * 4, axis=1)
        m = m2 * attc
        for ci in range(4):
            m5_ref[ci] = m[:, 128 * ci:128 * (ci + 1)]
        cfeat = jax.nn.silu(_mm(m, wc1_ref[...]) + bc1_ref[...])
        s = _mm(cfeat, wc2_ref[...])
        lane = lax.broadcasted_iota(jnp.int32, d.shape, 1)
        m5_ref[4] = d * s + jnp.where(lane == 3, 1.0, 0.0)

    full = lambda shape: pl.BlockSpec(shape, lambda i: (0,) * len(shape))
    return pl.pallas_call(
        body,
        grid=(E // TE,),
        in_specs=[
            pl.BlockSpec((TE, HID), lambda i: (i, 0)),
            pl.BlockSpec((TE, 128), lambda i: (i, 4)),
            pl.BlockSpec((TE, HID), lambda i: (i, 0)),
            pl.BlockSpec((TE, 128), lambda i: (i, 4)),
            pl.BlockSpec((TE, EDGE_NF), lambda i: (i, 0)),
            full((HID, 8)),
            full((HID, EDGE_NF)),
            full((1, HID)),
            full((HID, HID)),
            full((1, HID)),
            full((128, HID)),
            full((1, 128)),
            full((HID, HID)),
            full((1, HID)),
            full((128, HID)),
        ],
        out_specs=[pl.BlockSpec((5, TE, 128), lambda i: (0, i, 0))],
        out_shape=[jax.ShapeDtypeStruct((5, E, 128), F32)],
    )(g1, g1, g2, g2, efeats, wrad8, we1ef, be1, we2, be2, wa_rep, ba128,
      wc1, bc1, wc2_rep)[0]


def _sc_scatter(m5, row):
    EPS = E // NS
    BS = 80
    NB = EPS // BS
    NZ = 128
    RPT = NP // NS
    mesh = plsc.VectorSubcoreMesh(core_axis_name="c", subcore_axis_name="s")

    @functools.partial(
        pl.kernel,
        mesh=mesh,
        out_type=jax.ShapeDtypeStruct((5, NP, 128), F32),
        scratch_types=[
            pltpu.VMEM((BS, 128), F32),
            pltpu.VMEM((BS,), jnp.int32),
            pltpu.VMEM((NZ, 128), F32),
            pltpu.VMEM_SHARED((NP, 128), F32),
        ],
    )
    def k(m5_h, row_h, agg_h, mb, ib, zb, acc):
        c = lax.axis_index("c")
        s = lax.axis_index("s")

        @pl.loop(0, NZ)
        def _(r):
            @pl.loop(0, 8)
            def _(j):
                zb[r, pl.ds(j * 16, 16)] = jnp.zeros((16,), F32)

        for ch_i in range(3):
            last = ch_i == 2
            ch = 4 if last else c * 2 + ch_i

            @pl.loop(0, RPT // NZ)
            def _(z):
                pltpu.sync_copy(zb, acc.at[pl.ds(s * RPT + z * NZ, NZ)])

            plsc.subcore_barrier()

            def do_scatter(ch=ch):
                @pl.loop(0, NB)
                def _(b):
                    base = s * EPS + b * BS
                    pltpu.sync_copy(row_h.at[pl.ds(base, BS)], ib)
                    pltpu.sync_copy(m5_h.at[ch, pl.ds(base, BS)], mb)
                    pltpu.sync_copy(mb, acc.at[ib], add=True)

            if last:
                pl.when(c == 0)(do_scatter)
            else:
                do_scatter()

            plsc.subcore_barrier()

            def do_write(ch=ch):
                @pl.loop(0, RPT // NZ)
                def _(z):
                    r0 = s * RPT + z * NZ
                    pltpu.sync_copy(acc.at[pl.ds(r0, NZ)],
                                    agg_h.at[ch, pl.ds(r0, NZ)])

            if last:
                pl.when(c == 0)(do_write)
            else:
                do_write()

            plsc.subcore_barrier()

    return k(m5, row)


def _tc_node(hn, agg5, coordp, inp, wn1a, bn1, wn2, bn2, w_out, b_out):
    TN = 2000

    def body(hn_ref, a5_ref, cp_ref, x_ref,
             wn1a_ref, bn1_ref, wn2_ref, bn2_ref, wo_ref, bo_ref,
             out_ref, co_ref):
        wn1a = wn1a_ref[...]
        pre = hn_ref[...] + bn1_ref[...]
        for ci in range(4):
            pre = pre + _mm(a5_ref[ci], wn1a[:, 128 * ci:128 * (ci + 1)])
        nh = jax.nn.silu(pre)
        h2 = _mm(nh, wn2_ref[...]) + bn2_ref[...]
        out_ref[...] = _mm(h2, wo_ref[...]) + bo_ref[...] + x_ref[...]
        at = a5_ref[4][:, 0:16]
        rcp = 1.0 / jnp.maximum(at[:, 3:4], 1.0)
        r16 = _mm(rcp, jnp.ones((16, 1), F32))
        co_ref[...] = cp_ref[...] + at * r16

    full = lambda shape: pl.BlockSpec(shape, lambda i: (0,) * len(shape))
    return pl.pallas_call(
        body,
        grid=(N // TN,),
        in_specs=[
            pl.BlockSpec((TN, HID), lambda i: (i, 0)),
            pl.BlockSpec((5, TN, 128), lambda i: (0, i, 0)),
            pl.BlockSpec((TN, 16), lambda i: (i, 0)),
            pl.BlockSpec((TN, IN_SIZE), lambda i: (i, 0)),
            full((HID, HID)),
            full((1, HID)),
            full((HID, HID)),
            full((1, HID)),
            full((IN_SIZE, HID)),
            full((1, IN_SIZE)),
        ],
        out_specs=[
            pl.BlockSpec((TN, IN_SIZE), lambda i: (i, 0)),
            pl.BlockSpec((TN, 16), lambda i: (i, 0)),
        ],
        out_shape=[
            jax.ShapeDtypeStruct((N, IN_SIZE), F32),
            jax.ShapeDtypeStruct((N, 16), F32),
        ],
    )(hn, agg5, coordp, inp, wn1a, bn1, wn2, bn2, w_out, b_out)


def kernel(input, coord_feat, h0, lamda, alpha, l, edge_index, efeats,
           W_emb_in, b_emb_in, We1, be1, We2, be2, Wn1, bn1, Wn2, bn2,
           Wc1, bc1, Wc2, Wa, ba, W_emb_out, b_emb_out):
    row = edge_index[0].astype(jnp.int32)
    col = edge_index[1].astype(jnp.int32)
    coordp = jnp.pad(coord_feat, ((0, 0), (0, 13)))

    we1r = We1[:, :HID]
    we1c = We1[:, HID:2 * HID]
    wrad8 = jnp.tile(We1[:, 2 * HID].reshape(HID, 1), (1, 8))
    we1ef = We1[:, 2 * HID + 1:]
    wn1h = Wn1[:, :HID]
    wn1a = Wn1[:, HID:]
    wa_rep = jnp.tile(Wa, (128, 1))
    ba128 = jnp.tile(ba.reshape(1, 1), (1, 128))
    wc2_rep = jnp.tile(Wc2, (128, 1))
    r2 = lambda v: v.reshape(1, -1)

    hse, hce, hn = _tc_precompute(input, coordp, W_emb_in, r2(b_emb_in),
                                  we1r, we1c, wn1h)
    g1, g2 = _sc_gather(hse, hce, row, col)
    m5 = _tc_edge(g1, g2, efeats, wrad8, we1ef, r2(be1),
                  We2, r2(be2), wa_rep, ba128, Wc1, r2(bc1), wc2_rep)
    agg5 = _sc_scatter(m5, row)
    out, co16 = _tc_node(hn, agg5, coordp, input, wn1a,
                         r2(bn1), Wn2, r2(bn2), W_emb_out,
                         r2(b_emb_out))
    return out, co16[:, 0:3]


# speedup vs baseline: 2.6998x; 1.0667x over previous
import functools

import jax
import jax.numpy as jnp
from jax import lax
from jax.experimental import pallas as pl
from jax.experimental.pallas import tpu as pltpu
from jax.experimental.pallas import tpu_sc as plsc

N = 10000
NP = 10240
E = 160000
IN_SIZE = 256
HID = 512
EDGE_NF = 16
GW = HID + 128

NC = 2
NS = 16
NW = NC * NS

F32 = jnp.float32


def _mm(x, w):
    return lax.dot_general(x, w, (((1,), (1,)), ((), ())),
                           preferred_element_type=F32)


def _tc_precompute(inp, coordp, w_emb, b_emb, we1r, we1c, wn1h):
    TN = 2000

    def body(x_ref, cp_ref, wemb_ref, bemb_ref, wr_ref, wc_ref, wn_ref,
             hs_ref, hc_ref, hn_ref):
        h = _mm(x_ref[...], wemb_ref[...]) + bemb_ref[...]
        z = jnp.zeros((h.shape[0], GW - HID - 16), F32)
        cp = cp_ref[...]
        hs_ref[...] = jnp.concatenate([_mm(h, wr_ref[...]), cp, z], axis=1)
        hc_ref[...] = jnp.concatenate([_mm(h, wc_ref[...]), cp, z], axis=1)
        hn_ref[...] = _mm(h, wn_ref[...])

    full = lambda shape: pl.BlockSpec(shape, lambda i: (0,) * len(shape))
    return pl.pallas_call(
        body,
        grid=(N // TN,),
        in_specs=[
            pl.BlockSpec((TN, IN_SIZE), lambda i: (i, 0)),
            pl.BlockSpec((TN, 16), lambda i: (i, 0)),
            full((HID, IN_SIZE)),
            full((1, HID)),
            full((HID, HID)),
            full((HID, HID)),
            full((HID, HID)),
        ],
        out_specs=[
            pl.BlockSpec((TN, GW), lambda i: (i, 0)),
            pl.BlockSpec((TN, GW), lambda i: (i, 0)),
            pl.BlockSpec((TN, HID), lambda i: (i, 0)),
        ],
        out_shape=[
            jax.ShapeDtypeStruct((N, GW), F32),
            jax.ShapeDtypeStruct((N, GW), F32),
            jax.ShapeDtypeStruct((N, HID), F32),
        ],
    )(inp, coordp, w_emb, b_emb, we1r, we1c, wn1h)


def _sc_gather(hse, hce, row, col):
    EPS = E // NS
    B = 40
    NPAIR = EPS // (2 * B)
    mesh = plsc.VectorSubcoreMesh(core_axis_name="c", subcore_axis_name="s")

    @functools.partial(
        pl.kernel,
        mesh=mesh,
        out_type=(
            jax.ShapeDtypeStruct((E, GW), F32),
            jax.ShapeDtypeStruct((E, GW), F32),
        ),
        scratch_types=[
            pltpu.VMEM((B,), jnp.int32),
            pltpu.VMEM((B,), jnp.int32),
            pltpu.VMEM((B, GW), F32),
            pltpu.VMEM((B, GW), F32),
            pltpu.SemaphoreType.DMA,
            pltpu.SemaphoreType.DMA,
            pltpu.SemaphoreType.DMA,
            pltpu.SemaphoreType.DMA,
        ],
    )
    def k(hs_h, hc_h, row_h, col_h, g1_h, g2_h,
          iv0, iv1, gb0, gb1, sg0, sg1, sw0, sw1):
        c = lax.axis_index("c")
        s = lax.axis_index("s")
        iv = [iv0, iv1]
        gb = [gb0, gb1]
        sg = [sg0, sg1]
        sw = [sw0, sw1]

        def one_side(src_h, idx_h, out_h):
            @pl.loop(0, NPAIR)
            def pair(bb):
                for p in range(2):
                    base = s * EPS + (bb * 2 + p) * B

                    @pl.when(bb >= 1)
                    def wait_write(base=base, p=p):
                        pltpu.make_async_copy(
                            gb[p], out_h.at[pl.ds(base - 2 * B, B)],
                            sw[p]).wait()

                    pltpu.sync_copy(idx_h.at[pl.ds(base, B)], iv[p])
                    pltpu.async_copy(src_h.at[iv[p]], gb[p], sg[p])
                for p in range(2):
                    base = s * EPS + (bb * 2 + p) * B
                    pltpu.make_async_copy(src_h.at[iv[p]], gb[p],
                                          sg[p]).wait()
                    pltpu.async_copy(gb[p], out_h.at[pl.ds(base, B)], sw[p])

            for p in range(2):
                base = s * EPS + ((NPAIR - 1) * 2 + p) * B
                pltpu.make_async_copy(gb[p], out_h.at[pl.ds(base, B)],
                                      sw[p]).wait()

        @pl.when(c == 0)
        def side1():
            one_side(hs_h, row_h, g1_h)

        @pl.when(c == 1)
        def side2():
            one_side(hc_h, col_h, g2_h)

    return k(hse, hce, row, col)


def _tc_edge(g1, g2, efeats, wrad8, we1ef, be1, we2, be2,
             wa_rep, ba128, wc1, bc1, wc2_rep):
    TE = 1280

    def body(g1h_ref, g1c_ref, g2h_ref, g2c_ref, ef_ref,
             wrad8_ref, we1ef_ref, be1_ref, we2_ref, be2_ref,
             wa_ref, ba_ref, wc1_ref, bc1_ref, wc2_ref,
             m5_ref):
        d = g1c_ref[...] - g2c_ref[...]
        d8 = d[:, 0:8]
        pre = (g1h_ref[...] + g2h_ref[...] + _mm(d8 * d8, wrad8_ref[...])
               + _mm(ef_ref[...], we1ef_ref[...]) + be1_ref[...])
        m1 = jax.nn.silu(pre)
        m2 = jax.nn.silu(_mm(m1, we2_ref[...]) + be2_ref[...])
        att = jax.nn.sigmoid(_mm(m2, wa_ref[...]) + ba_ref[...])
        attc = jnp.concatenate([att] * 4, axis=1)
        m = m2 * attc
        for ci in range(4):
            m5_ref[ci] = m[:, 128 * ci:128 * (ci + 1)]
        cfeat = jax.nn.silu(_mm(m, wc1_ref[...]) + bc1_ref[...])
        s = _mm(cfeat, wc2_ref[...])
        lane = lax.broadcasted_iota(jnp.int32, d.shape, 1)
        m5_ref[4] = d * s + jnp.where(lane == 3, 1.0, 0.0)

    full = lambda shape: pl.BlockSpec(shape, lambda i: (0,) * len(shape))
    return pl.pallas_call(
        body,
        grid=(E // TE,),
        in_specs=[
            pl.BlockSpec((TE, HID), lambda i: (i, 0)),
            pl.BlockSpec((TE, 128), lambda i: (i, 4)),
            pl.BlockSpec((TE, HID), lambda i: (i, 0)),
            pl.BlockSpec((TE, 128), lambda i: (i, 4)),
            pl.BlockSpec((TE, EDGE_NF), lambda i: (i, 0)),
            full((HID, 8)),
            full((HID, EDGE_NF)),
            full((1, HID)),
            full((HID, HID)),
            full((1, HID)),
            full((128, HID)),
            full((1, 128)),
            full((HID, HID)),
            full((1, HID)),
            full((128, HID)),
        ],
        out_specs=[pl.BlockSpec((5, TE, 128), lambda i: (0, i, 0))],
        out_shape=[jax.ShapeDtypeStruct((5, E, 128), F32)],
    )(g1, g1, g2, g2, efeats, wrad8, we1ef, be1, we2, be2, wa_rep, ba128,
      wc1, bc1, wc2_rep)[0]


def _sc_scatter(m5, row):
    EPS = E // NS
    BS = 80
    NB = EPS // BS
    NZ = 128
    RPT = NP // NS
    mesh = plsc.VectorSubcoreMesh(core_axis_name="c", subcore_axis_name="s")

    @functools.partial(
        pl.kernel,
        mesh=mesh,
        out_type=jax.ShapeDtypeStruct((5, NP, 128), F32),
        scratch_types=[
            pltpu.VMEM((BS, 128), F32),
            pltpu.VMEM((BS,), jnp.int32),
            pltpu.VMEM((NZ, 128), F32),
            pltpu.VMEM_SHARED((NP, 128), F32),
        ],
    )
    def k(m5_h, row_h, agg_h, mb, ib, zb, acc):
        c = lax.axis_index("c")
        s = lax.axis_index("s")

        @pl.loop(0, NZ)
        def _(r):
            @pl.loop(0, 8)
            def _(j):
                zb[r, pl.ds(j * 16, 16)] = jnp.zeros((16,), F32)

        for ch_i in range(3):
            last = ch_i == 2
            ch = 4 if last else c * 2 + ch_i

            @pl.loop(0, RPT // NZ)
            def _(z):
                pltpu.sync_copy(zb, acc.at[pl.ds(s * RPT + z * NZ, NZ)])

            plsc.subcore_barrier()

            def do_scatter(ch=ch):
                @pl.loop(0, NB)
                def _(b):
                    base = s * EPS + b * BS
                    pltpu.sync_copy(row_h.at[pl.ds(base, BS)], ib)
                    pltpu.sync_copy(m5_h.at[ch, pl.ds(base, BS)], mb)
                    pltpu.sync_copy(mb, acc.at[ib], add=True)

            if last:
                pl.when(c == 0)(do_scatter)
            else:
                do_scatter()

            plsc.subcore_barrier()

            def do_write(ch=ch):
                @pl.loop(0, RPT // NZ)
                def _(z):
                    r0 = s * RPT + z * NZ
                    pltpu.sync_copy(acc.at[pl.ds(r0, NZ)],
                                    agg_h.at[ch, pl.ds(r0, NZ)])

            if last:
                pl.when(c == 0)(do_write)
            else:
                do_write()

            plsc.subcore_barrier()

    return k(m5, row)


def _tc_node(hn, agg5, coordp, inp, wn1a, bn1, wn2, bn2, w_out, b_out):
    TN = 2000

    def body(hn_ref, a5_ref, cp_ref, x_ref,
             wn1a_ref, bn1_ref, wn2_ref, bn2_ref, wo_ref, bo_ref,
             out_ref, co_ref):
        wn1a = wn1a_ref[...]
        pre = hn_ref[...] + bn1_ref[...]
        for ci in range(4):
            pre = pre + _mm(a5_ref[ci], wn1a[:, 128 * ci:128 * (ci + 1)])
        nh = jax.nn.silu(pre)
        h2 = _mm(nh, wn2_ref[...]) + bn2_ref[...]
        out_ref[...] = _mm(h2, wo_ref[...]) + bo_ref[...] + x_ref[...]
        at = a5_ref[4][:, 0:16]
        rcp = 1.0 / jnp.maximum(at[:, 3:4], 1.0)
        r16 = _mm(rcp, jnp.ones((16, 1), F32))
        co_ref[...] = cp_ref[...] + at * r16

    full = lambda shape: pl.BlockSpec(shape, lambda i: (0,) * len(shape))
    return pl.pallas_call(
        body,
        grid=(N // TN,),
        in_specs=[
            pl.BlockSpec((TN, HID), lambda i: (i, 0)),
            pl.BlockSpec((5, TN, 128), lambda i: (0, i, 0)),
            pl.BlockSpec((TN, 16), lambda i: (i, 0)),
            pl.BlockSpec((TN, IN_SIZE), lambda i: (i, 0)),
            full((HID, HID)),
            full((1, HID)),
            full((HID, HID)),
            full((1, HID)),
            full((IN_SIZE, HID)),
            full((1, IN_SIZE)),
        ],
        out_specs=[
            pl.BlockSpec((TN, IN_SIZE), lambda i: (i, 0)),
            pl.BlockSpec((TN, 16), lambda i: (i, 0)),
        ],
        out_shape=[
            jax.ShapeDtypeStruct((N, IN_SIZE), F32),
            jax.ShapeDtypeStruct((N, 16), F32),
        ],
    )(hn, agg5, coordp, inp, wn1a, bn1, wn2, bn2, w_out, b_out)


def kernel(input, coord_feat, h0, lamda, alpha, l, edge_index, efeats,
           W_emb_in, b_emb_in, We1, be1, We2, be2, Wn1, bn1, Wn2, bn2,
           Wc1, bc1, Wc2, Wa, ba, W_emb_out, b_emb_out):
    row = edge_index[0].astype(jnp.int32)
    col = edge_index[1].astype(jnp.int32)
    coordp = jnp.pad(coord_feat, ((0, 0), (0, 13)))

    we1r = We1[:, :HID]
    we1c = We1[:, HID:2 * HID]
    wrad8 = jnp.tile(We1[:, 2 * HID].reshape(HID, 1), (1, 8))
    we1ef = We1[:, 2 * HID + 1:]
    wn1h = Wn1[:, :HID]
    wn1a = Wn1[:, HID:]
    wa_rep = jnp.tile(Wa, (128, 1))
    ba128 = jnp.tile(ba.reshape(1, 1), (1, 128))
    wc2_rep = jnp.tile(Wc2, (128, 1))
    r2 = lambda v: v.reshape(1, -1)

    hse, hce, hn = _tc_precompute(input, coordp, W_emb_in, r2(b_emb_in),
                                  we1r, we1c, wn1h)
    g1, g2 = _sc_gather(hse, hce, row, col)
    m5 = _tc_edge(g1, g2, efeats, wrad8, we1ef, r2(be1),
                  We2, r2(be2), wa_rep, ba128, Wc1, r2(bc1), wc2_rep)
    agg5 = _sc_scatter(m5, row)
    out, co16 = _tc_node(hn, agg5, coordp, input, wn1a,
                         r2(bn1), Wn2, r2(bn2), W_emb_out,
                         r2(b_emb_out))
    return out, co16[:, 0:3]


# speedup vs baseline: 3.4787x; 1.2885x over previous
import functools

import jax
import jax.numpy as jnp
from jax import lax
from jax.experimental import pallas as pl
from jax.experimental.pallas import tpu as pltpu
from jax.experimental.pallas import tpu_sc as plsc

N = 10000
NP = 10240
E = 160000
IN_SIZE = 256
HID = 512
EDGE_NF = 16
GW = HID + 128

NC = 2
NS = 16
NW = NC * NS

F32 = jnp.float32


def _mm(x, w):
    return lax.dot_general(x, w, (((1,), (1,)), ((), ())),
                           preferred_element_type=F32)


def _tc_precompute(inp, coordp, w_emb, b_emb, we1r, we1c, wn1h):
    TN = 2000

    def body(x_ref, cp_ref, wemb_ref, bemb_ref, wr_ref, wc_ref, wn_ref,
             hs_ref, hc_ref, hn_ref):
        h = _mm(x_ref[...], wemb_ref[...]) + bemb_ref[...]
        z = jnp.zeros((h.shape[0], GW - HID - 16), F32)
        cp = cp_ref[...]
        hs_ref[...] = jnp.concatenate([_mm(h, wr_ref[...]), cp, z], axis=1)
        hc_ref[...] = jnp.concatenate([_mm(h, wc_ref[...]), cp, z], axis=1)
        hn_ref[...] = _mm(h, wn_ref[...])

    full = lambda shape: pl.BlockSpec(shape, lambda i: (0,) * len(shape))
    return pl.pallas_call(
        body,
        grid=(N // TN,),
        in_specs=[
            pl.BlockSpec((TN, IN_SIZE), lambda i: (i, 0)),
            pl.BlockSpec((TN, 16), lambda i: (i, 0)),
            full((HID, IN_SIZE)),
            full((1, HID)),
            full((HID, HID)),
            full((HID, HID)),
            full((HID, HID)),
        ],
        out_specs=[
            pl.BlockSpec((TN, GW), lambda i: (i, 0)),
            pl.BlockSpec((TN, GW), lambda i: (i, 0)),
            pl.BlockSpec((TN, HID), lambda i: (i, 0)),
        ],
        out_shape=[
            jax.ShapeDtypeStruct((N, GW), F32),
            jax.ShapeDtypeStruct((N, GW), F32),
            jax.ShapeDtypeStruct((N, HID), F32),
        ],
    )(inp, coordp, w_emb, b_emb, we1r, we1c, wn1h)


def _sc_gather(hse, hce, row, col):
    EPS = E // NS
    B = 40
    NPAIR = EPS // (2 * B)
    mesh = plsc.VectorSubcoreMesh(core_axis_name="c", subcore_axis_name="s")

    @functools.partial(
        pl.kernel,
        mesh=mesh,
        out_type=(
            jax.ShapeDtypeStruct((E, GW), F32),
            jax.ShapeDtypeStruct((E, GW), F32),
        ),
        scratch_types=[
            pltpu.VMEM((B,), jnp.int32),
            pltpu.VMEM((B,), jnp.int32),
            pltpu.VMEM((B, GW), F32),
            pltpu.VMEM((B, GW), F32),
            pltpu.SemaphoreType.DMA,
            pltpu.SemaphoreType.DMA,
            pltpu.SemaphoreType.DMA,
            pltpu.SemaphoreType.DMA,
        ],
    )
    def k(hs_h, hc_h, row_h, col_h, g1_h, g2_h,
          iv0, iv1, gb0, gb1, sg0, sg1, sw0, sw1):
        c = lax.axis_index("c")
        s = lax.axis_index("s")
        iv = [iv0, iv1]
        gb = [gb0, gb1]
        sg = [sg0, sg1]
        sw = [sw0, sw1]

        def one_side(src_h, idx_h, out_h):
            @pl.loop(0, NPAIR)
            def pair(bb):
                for p in range(2):
                    base = s * EPS + (bb * 2 + p) * B

                    @pl.when(bb >= 1)
                    def wait_write(base=base, p=p):
                        pltpu.make_async_copy(
                            gb[p], out_h.at[pl.ds(base - 2 * B, B)],
                            sw[p]).wait()

                    pltpu.sync_copy(idx_h.at[pl.ds(base, B)], iv[p])
                    pltpu.async_copy(src_h.at[iv[p]], gb[p], sg[p])
                for p in range(2):
                    base = s * EPS + (bb * 2 + p) * B
                    pltpu.make_async_copy(src_h.at[iv[p]], gb[p],
                                          sg[p]).wait()
                    pltpu.async_copy(gb[p], out_h.at[pl.ds(base, B)], sw[p])

            for p in range(2):
                base = s * EPS + ((NPAIR - 1) * 2 + p) * B
                pltpu.make_async_copy(gb[p], out_h.at[pl.ds(base, B)],
                                      sw[p]).wait()

        @pl.when(c == 0)
        def side1():
            one_side(hs_h, row_h, g1_h)

        @pl.when(c == 1)
        def side2():
            one_side(hc_h, col_h, g2_h)

    return k(hse, hce, row, col)


def _tc_edge(g1, g2, efeats, wrad8, we1ef, be1, we2, be2,
             wa_rep, ba128, wc1, bc1, wc2_rep):
    TE = 1280

    def body(g1h_ref, g1c_ref, g2h_ref, g2c_ref, ef_ref,
             wrad8_ref, we1ef_ref, be1_ref, we2_ref, be2_ref,
             wa_ref, ba_ref, wc1_ref, bc1_ref, wc2_ref,
             m5_ref):
        d = g1c_ref[...] - g2c_ref[...]
        d8 = d[:, 0:8]
        pre = (g1h_ref[...] + g2h_ref[...] + _mm(d8 * d8, wrad8_ref[...])
               + _mm(ef_ref[...], we1ef_ref[...]) + be1_ref[...])
        m1 = jax.nn.silu(pre)
        m2 = jax.nn.silu(_mm(m1, we2_ref[...]) + be2_ref[...])
        att = jax.nn.sigmoid(_mm(m2, wa_ref[...]) + ba_ref[...])
        attc = jnp.concatenate([att] * 4, axis=1)
        m = m2 * attc
        for ci in range(4):
            m5_ref[ci] = m[:, 128 * ci:128 * (ci + 1)]
        cfeat = jax.nn.silu(_mm(m, wc1_ref[...]) + bc1_ref[...])
        s = _mm(cfeat, wc2_ref[...])
        lane = lax.broadcasted_iota(jnp.int32, d.shape, 1)
        m5_ref[4] = d * s + jnp.where(lane == 3, 1.0, 0.0)

    full = lambda shape: pl.BlockSpec(shape, lambda i: (0,) * len(shape))
    return pl.pallas_call(
        body,
        grid=(E // TE,),
        in_specs=[
            pl.BlockSpec((TE, HID), lambda i: (i, 0)),
            pl.BlockSpec((TE, 128), lambda i: (i, 4)),
            pl.BlockSpec((TE, HID), lambda i: (i, 0)),
            pl.BlockSpec((TE, 128), lambda i: (i, 4)),
            pl.BlockSpec((TE, EDGE_NF), lambda i: (i, 0)),
            full((HID, 8)),
            full((HID, EDGE_NF)),
            full((1, HID)),
            full((HID, HID)),
            full((1, HID)),
            full((128, HID)),
            full((1, 128)),
            full((HID, HID)),
            full((1, HID)),
            full((128, HID)),
        ],
        out_specs=[pl.BlockSpec((5, TE, 128), lambda i: (0, i, 0))],
        out_shape=[jax.ShapeDtypeStruct((5, E, 128), F32)],
    )(g1, g1, g2, g2, efeats, wrad8, we1ef, be1, we2, be2, wa_rep, ba128,
      wc1, bc1, wc2_rep)[0]


def _sc_scatter(m5, row):
    EPS = E // NS
    BS = 80
    NZ = 128
    RPT = NP // NS
    mesh = plsc.VectorSubcoreMesh(core_axis_name="c", subcore_axis_name="s")

    @functools.partial(
        pl.kernel,
        mesh=mesh,
        out_type=jax.ShapeDtypeStruct((6, NP, 128), F32),
        scratch_types=[
            pltpu.VMEM((BS,), jnp.int32),
            pltpu.VMEM((BS,), jnp.int32),
            pltpu.VMEM((BS // 2,), jnp.int32),
            pltpu.VMEM((BS // 2,), jnp.int32),
            pltpu.VMEM((BS, 128), F32),
            pltpu.VMEM((BS, 128), F32),
            pltpu.VMEM((BS // 2, 128), F32),
            pltpu.VMEM((BS // 2, 128), F32),
            pltpu.VMEM((NZ, 128), F32),
            pltpu.VMEM_SHARED((NP, 128), F32),
            pltpu.SemaphoreType.DMA,
            pltpu.SemaphoreType.DMA,
            pltpu.SemaphoreType.DMA,
            pltpu.SemaphoreType.DMA,
        ],
    )
    def k(m5_h, row_h, agg_h, ib0, ib1, jb0, jb1, mb0, mb1, nb0, nb1,
          zb, acc, si0, si1, sm0, sm1):
        c = lax.axis_index("c")
        s = lax.axis_index("s")
        si = [si0, si1]
        sm = [sm0, sm1]

        @pl.loop(0, NZ)
        def zr(r):
            @pl.loop(0, 8)
            def zc(j):
                zb[r, pl.ds(j * 16, 16)] = jnp.zeros((16,), F32)

        def zero_acc():
            @pl.loop(0, RPT // NZ)
            def zz(z):
                pltpu.sync_copy(zb, acc.at[pl.ds(s * RPT + z * NZ, NZ)])

        def writeout(plane):
            @pl.loop(0, RPT // NZ)
            def wz(z):
                r0 = s * RPT + z * NZ
                pltpu.sync_copy(acc.at[pl.ds(r0, NZ)],
                                agg_h.at[plane, pl.ds(r0, NZ)])

        def sweep(ch, first, eps, bs, ib, mb):
            nb = eps // bs
            npair = (nb - 1) // 2

            def start(b, p):
                base = first + b * bs
                pltpu.async_copy(row_h.at[pl.ds(base, bs)], ib[p], si[p])
                pltpu.async_copy(m5_h.at[ch, pl.ds(base, bs)], mb[p], sm[p])

            def flush(p):
                pltpu.make_async_copy(row_h.at[pl.ds(first, bs)],
                                      ib[p], si[p]).wait()
                pltpu.make_async_copy(m5_h.at[ch, pl.ds(first, bs)],
                                      mb[p], sm[p]).wait()
                pltpu.sync_copy(mb[p], acc.at[ib[p]], add=True)

            start(0, 0)

            @pl.loop(0, npair)
            def pr(bb):
                for (p, off) in ((1, 1), (0, 2)):
                    start(2 * bb + off, p)
                    flush(1 - p)

            flush(0)

        for ch_i in range(2):
            zero_acc()
            plsc.subcore_barrier()
            sweep(c * 2 + ch_i, s * EPS, EPS, BS, [ib0, ib1], [mb0, mb1])
            plsc.subcore_barrier()
            writeout(c * 2 + ch_i)
            plsc.subcore_barrier()

        zero_acc()
        plsc.subcore_barrier()
        sweep(4, c * (E // 2) + s * (EPS // 2), EPS // 2, BS // 2,
              [jb0, jb1], [nb0, nb1])
        plsc.subcore_barrier()
        writeout(4 + c)
        plsc.subcore_barrier()

    return k(m5, row)


def _tc_node(hn, agg5, coordp, inp, wn1a, bn1, wn2, bn2, w_out, b_out):
    TN = 2000

    def body(hn_ref, a5_ref, cp_ref, x_ref,
             wn1a_ref, bn1_ref, wn2_ref, bn2_ref, wo_ref, bo_ref,
             out_ref, co_ref):
        wn1a = wn1a_ref[...]
        pre = hn_ref[...] + bn1_ref[...]
        for ci in range(4):
            pre = pre + _mm(a5_ref[ci], wn1a[:, 128 * ci:128 * (ci + 1)])
        nh = jax.nn.silu(pre)
        h2 = _mm(nh, wn2_ref[...]) + bn2_ref[...]
        out_ref[...] = _mm(h2, wo_ref[...]) + bo_ref[...] + x_ref[...]
        at = a5_ref[4][:, 0:16] + a5_ref[5][:, 0:16]
        rcp = 1.0 / jnp.maximum(at[:, 3:4], 1.0)
        r16 = _mm(rcp, jnp.ones((16, 1), F32))
        co_ref[...] = cp_ref[...] + at * r16

    full = lambda shape: pl.BlockSpec(shape, lambda i: (0,) * len(shape))
    return pl.pallas_call(
        body,
        grid=(N // TN,),
        in_specs=[
            pl.BlockSpec((TN, HID), lambda i: (i, 0)),
            pl.BlockSpec((6, TN, 128), lambda i: (0, i, 0)),
            pl.BlockSpec((TN, 16), lambda i: (i, 0)),
            pl.BlockSpec((TN, IN_SIZE), lambda i: (i, 0)),
            full((HID, HID)),
            full((1, HID)),
            full((HID, HID)),
            full((1, HID)),
            full((IN_SIZE, HID)),
            full((1, IN_SIZE)),
        ],
        out_specs=[
            pl.BlockSpec((TN, IN_SIZE), lambda i: (i, 0)),
            pl.BlockSpec((TN, 16), lambda i: (i, 0)),
        ],
        out_shape=[
            jax.ShapeDtypeStruct((N, IN_SIZE), F32),
            jax.ShapeDtypeStruct((N, 16), F32),
        ],
    )(hn, agg5, coordp, inp, wn1a, bn1, wn2, bn2, w_out, b_out)


def kernel(input, coord_feat, h0, lamda, alpha, l, edge_index, efeats,
           W_emb_in, b_emb_in, We1, be1, We2, be2, Wn1, bn1, Wn2, bn2,
           Wc1, bc1, Wc2, Wa, ba, W_emb_out, b_emb_out):
    row = edge_index[0].astype(jnp.int32)
    col = edge_index[1].astype(jnp.int32)
    coordp = jnp.pad(coord_feat, ((0, 0), (0, 13)))

    we1r = We1[:, :HID]
    we1c = We1[:, HID:2 * HID]
    wrad8 = jnp.tile(We1[:, 2 * HID].reshape(HID, 1), (1, 8))
    we1ef = We1[:, 2 * HID + 1:]
    wn1h = Wn1[:, :HID]
    wn1a = Wn1[:, HID:]
    wa_rep = jnp.tile(Wa, (128, 1))
    ba128 = jnp.tile(ba.reshape(1, 1), (1, 128))
    wc2_rep = jnp.tile(Wc2, (128, 1))
    r2 = lambda v: v.reshape(1, -1)

    hse, hce, hn = _tc_precompute(input, coordp, W_emb_in, r2(b_emb_in),
                                  we1r, we1c, wn1h)
    g1, g2 = _sc_gather(hse, hce, row, col)
    m5 = _tc_edge(g1, g2, efeats, wrad8, we1ef, r2(be1),
                  We2, r2(be2), wa_rep, ba128, Wc1, r2(bc1), wc2_rep)
    agg5 = _sc_scatter(m5, row)
    out, co16 = _tc_node(hn, agg5, coordp, input, wn1a,
                         r2(bn1), Wn2, r2(bn2), W_emb_out,
                         r2(b_emb_out))
    return out, co16[:, 0:3]


# speedup vs baseline: 3.5470x; 1.0196x over previous
import functools

import jax
import jax.numpy as jnp
from jax import lax
from jax.experimental import pallas as pl
from jax.experimental.pallas import tpu as pltpu
from jax.experimental.pallas import tpu_sc as plsc

N = 10000
NP = 10240
E = 160000
EH = E // 2
IN_SIZE = 256
HID = 512
EDGE_NF = 16
GW = HID + 128

NC = 2
NS = 16
NW = NC * NS

F32 = jnp.float32


def _mm(x, w):
    return lax.dot_general(x, w, (((1,), (1,)), ((), ())),
                           preferred_element_type=F32)


def _tc_precompute(inp, coordp, w_emb, b_emb, we1r, we1c, wn1h):
    TN = 2000

    def body(x_ref, cp_ref, wemb_ref, bemb_ref, wr_ref, wc_ref, wn_ref,
             hs_ref, hc_ref, hn_ref):
        h = _mm(x_ref[...], wemb_ref[...]) + bemb_ref[...]
        z = jnp.zeros((h.shape[0], GW - HID - 16), F32)
        cp = cp_ref[...]
        hs_ref[...] = jnp.concatenate([_mm(h, wr_ref[...]), cp, z], axis=1)
        hc_ref[...] = jnp.concatenate([_mm(h, wc_ref[...]), cp, z], axis=1)
        hn_ref[...] = _mm(h, wn_ref[...])

    full = lambda shape: pl.BlockSpec(shape, lambda i: (0,) * len(shape))
    return pl.pallas_call(
        body,
        grid=(N // TN,),
        in_specs=[
            pl.BlockSpec((TN, IN_SIZE), lambda i: (i, 0)),
            pl.BlockSpec((TN, 16), lambda i: (i, 0)),
            full((HID, IN_SIZE)),
            full((1, HID)),
            full((HID, HID)),
            full((HID, HID)),
            full((HID, HID)),
        ],
        out_specs=[
            pl.BlockSpec((TN, GW), lambda i: (i, 0)),
            pl.BlockSpec((TN, GW), lambda i: (i, 0)),
            pl.BlockSpec((TN, HID), lambda i: (i, 0)),
        ],
        out_shape=[
            jax.ShapeDtypeStruct((N, GW), F32),
            jax.ShapeDtypeStruct((N, GW), F32),
            jax.ShapeDtypeStruct((N, HID), F32),
        ],
    )(inp, coordp, w_emb, b_emb, we1r, we1c, wn1h)


def _sc_gather(hse, hce, row, col):
    EPS = EH // NS
    B = 40
    NB = EPS // B
    NPAIR = NB // 2
    mesh = plsc.VectorSubcoreMesh(core_axis_name="c", subcore_axis_name="s")

    @functools.partial(
        pl.kernel,
        mesh=mesh,
        out_type=(
            jax.ShapeDtypeStruct((EH, GW), F32),
            jax.ShapeDtypeStruct((EH, GW), F32),
        ),
        scratch_types=[
            pltpu.VMEM((B,), jnp.int32),
            pltpu.VMEM((B,), jnp.int32),
            pltpu.VMEM((B, GW), F32),
            pltpu.VMEM((B, GW), F32),
            pltpu.SemaphoreType.DMA,
            pltpu.SemaphoreType.DMA,
            pltpu.SemaphoreType.DMA,
            pltpu.SemaphoreType.DMA,
        ],
    )
    def k(hs_h, hc_h, row_h, col_h, g1_h, g2_h,
          iv0, iv1, gb0, gb1, sg0, sg1, sw0, sw1):
        c = lax.axis_index("c")
        s = lax.axis_index("s")
        iv = [iv0, iv1]
        gb = [gb0, gb1]
        sg = [sg0, sg1]
        sw = [sw0, sw1]

        def one_side(src_h, idx_h, out_h):
            @pl.loop(0, NPAIR)
            def pair(bb):
                for p in range(2):
                    base = s * EPS + (bb * 2 + p) * B

                    @pl.when(bb >= 1)
                    def wait_write(base=base, p=p):
                        pltpu.make_async_copy(
                            gb[p], out_h.at[pl.ds(base - 2 * B, B)],
                            sw[p]).wait()

                    pltpu.sync_copy(idx_h.at[pl.ds(base, B)], iv[p])
                    pltpu.async_copy(src_h.at[iv[p]], gb[p], sg[p])
                for p in range(2):
                    base = s * EPS + (bb * 2 + p) * B
                    pltpu.make_async_copy(src_h.at[iv[p]], gb[p],
                                          sg[p]).wait()
                    pltpu.async_copy(gb[p], out_h.at[pl.ds(base, B)], sw[p])

            for p in range(2):
                base = s * EPS + ((NPAIR - 1) * 2 + p) * B
                pltpu.make_async_copy(gb[p], out_h.at[pl.ds(base, B)],
                                      sw[p]).wait()
            base = s * EPS + (NB - 1) * B
            pltpu.sync_copy(idx_h.at[pl.ds(base, B)], iv[0])
            pltpu.async_copy(src_h.at[iv[0]], gb[0], sg[0])
            pltpu.make_async_copy(src_h.at[iv[0]], gb[0], sg[0]).wait()
            pltpu.sync_copy(gb[0], out_h.at[pl.ds(base, B)])

        @pl.when(c == 0)
        def side1():
            one_side(hs_h, row_h, g1_h)

        @pl.when(c == 1)
        def side2():
            one_side(hc_h, col_h, g2_h)

    return k(hse, hce, row, col)


def _tc_edge(g1, g2, efeats, wrad8, we1ef, be1, we2, be2,
             wa_rep, ba128, wc1, bc1, wc2_rep):
    TE = 1600

    def body(g1h_ref, g1c_ref, g2h_ref, g2c_ref, ef_ref,
             wrad8_ref, we1ef_ref, be1_ref, we2_ref, be2_ref,
             wa_ref, ba_ref, wc1_ref, bc1_ref, wc2_ref,
             m5_ref):
        d = g1c_ref[...] - g2c_ref[...]
        d8 = d[:, 0:8]
        pre = (g1h_ref[...] + g2h_ref[...] + _mm(d8 * d8, wrad8_ref[...])
               + _mm(ef_ref[...], we1ef_ref[...]) + be1_ref[...])
        m1 = jax.nn.silu(pre)
        m2 = jax.nn.silu(_mm(m1, we2_ref[...]) + be2_ref[...])
        att = jax.nn.sigmoid(_mm(m2, wa_ref[...]) + ba_ref[...])
        attc = jnp.concatenate([att] * 4, axis=1)
        m = m2 * attc
        for ci in range(4):
            m5_ref[ci] = m[:, 128 * ci:128 * (ci + 1)]
        cfeat = jax.nn.silu(_mm(m, wc1_ref[...]) + bc1_ref[...])
        s = _mm(cfeat, wc2_ref[...])
        lane = lax.broadcasted_iota(jnp.int32, d.shape, 1)
        m5_ref[4] = d * s + jnp.where(lane == 3, 1.0, 0.0)

    full = lambda shape: pl.BlockSpec(shape, lambda i: (0,) * len(shape))
    return pl.pallas_call(
        body,
        grid=(EH // TE,),
        in_specs=[
            pl.BlockSpec((TE, HID), lambda i: (i, 0)),
            pl.BlockSpec((TE, 128), lambda i: (i, 4)),
            pl.BlockSpec((TE, HID), lambda i: (i, 0)),
            pl.BlockSpec((TE, 128), lambda i: (i, 4)),
            pl.BlockSpec((TE, EDGE_NF), lambda i: (i, 0)),
            full((HID, 8)),
            full((HID, EDGE_NF)),
            full((1, HID)),
            full((HID, HID)),
            full((1, HID)),
            full((128, HID)),
            full((1, 128)),
            full((HID, HID)),
            full((1, HID)),
            full((128, HID)),
        ],
        out_specs=[pl.BlockSpec((5, TE, 128), lambda i: (0, i, 0))],
        out_shape=[jax.ShapeDtypeStruct((5, EH, 128), F32)],
    )(g1, g1, g2, g2, efeats, wrad8, we1ef, be1, we2, be2, wa_rep, ba128,
      wc1, bc1, wc2_rep)[0]


def _sc_scatter(m5, row, c4_core):
    EPS = EH // NS
    BS = 40
    NZ = 128
    RPT = NP // NS
    mesh = plsc.VectorSubcoreMesh(core_axis_name="c", subcore_axis_name="s")

    @functools.partial(
        pl.kernel,
        mesh=mesh,
        out_type=jax.ShapeDtypeStruct((5, NP, 128), F32),
        scratch_types=[
            pltpu.VMEM((BS,), jnp.int32),
            pltpu.VMEM((BS,), jnp.int32),
            pltpu.VMEM((BS, 128), F32),
            pltpu.VMEM((BS, 128), F32),
            pltpu.VMEM((NZ, 128), F32),
            pltpu.VMEM_SHARED((NP, 128), F32),
            pltpu.SemaphoreType.DMA,
            pltpu.SemaphoreType.DMA,
            pltpu.SemaphoreType.DMA,
            pltpu.SemaphoreType.DMA,
        ],
    )
    def k(m5_h, row_h, agg_h, ib0, ib1, mb0, mb1,
          zb, acc, si0, si1, sm0, sm1):
        c = lax.axis_index("c")
        s = lax.axis_index("s")
        ib = [ib0, ib1]
        mb = [mb0, mb1]
        si = [si0, si1]
        sm = [sm0, sm1]

        @pl.loop(0, NZ)
        def zr(r):
            @pl.loop(0, 8)
            def zc(j):
                zb[r, pl.ds(j * 16, 16)] = jnp.zeros((16,), F32)

        def zero_acc():
            @pl.loop(0, RPT // NZ)
            def zz(z):
                pltpu.sync_copy(zb, acc.at[pl.ds(s * RPT + z * NZ, NZ)])

        def writeout(plane):
            @pl.loop(0, RPT // NZ)
            def wz(z):
                r0 = s * RPT + z * NZ
                pltpu.sync_copy(acc.at[pl.ds(r0, NZ)],
                                agg_h.at[plane, pl.ds(r0, NZ)])

        def sweep(ch):
            first = s * EPS
            nb = EPS // BS
            npair = (nb - 1) // 2

            def start(b, p):
                base = first + b * BS
                pltpu.async_copy(row_h.at[pl.ds(base, BS)], ib[p], si[p])
                pltpu.async_copy(m5_h.at[ch, pl.ds(base, BS)], mb[p], sm[p])

            def flush(p):
                pltpu.make_async_copy(row_h.at[pl.ds(first, BS)],
                                      ib[p], si[p]).wait()
                pltpu.make_async_copy(m5_h.at[ch, pl.ds(first, BS)],
                                      mb[p], sm[p]).wait()
                pltpu.sync_copy(mb[p], acc.at[ib[p]], add=True)

            start(0, 0)

            @pl.loop(0, npair)
            def pr(bb):
                for (p, off) in ((1, 1), (0, 2)):
                    start(2 * bb + off, p)
                    flush(1 - p)

            flush(0)

        for ch_i in range(2):
            zero_acc()
            plsc.subcore_barrier()
            sweep(c * 2 + ch_i)
            plsc.subcore_barrier()
            writeout(c * 2 + ch_i)
            plsc.subcore_barrier()

        zero_acc()
        plsc.subcore_barrier()

        @pl.when(c == c4_core)
        def c4_sweep():
            sweep(4)

        plsc.subcore_barrier()

        @pl.when(c == c4_core)
        def c4_write():
            writeout(4)

        plsc.subcore_barrier()

    return k(m5, row)


def _tc_node(hn, agg_a, agg_b, coordp, inp, wn1a, bn1, wn2, bn2,
             w_out, b_out):
    TN = 2000

    def body(hn_ref, aa_ref, ab_ref, cp_ref, x_ref,
             wn1a_ref, bn1_ref, wn2_ref, bn2_ref, wo_ref, bo_ref,
             out_ref, co_ref):
        wn1a = wn1a_ref[...]
        pre = hn_ref[...] + bn1_ref[...]
        for ci in range(4):
            agg = aa_ref[ci] + ab_ref[ci]
            pre = pre + _mm(agg, wn1a[:, 128 * ci:128 * (ci + 1)])
        nh = jax.nn.silu(pre)
        h2 = _mm(nh, wn2_ref[...]) + bn2_ref[...]
        out_ref[...] = _mm(h2, wo_ref[...]) + bo_ref[...] + x_ref[...]
        at = (aa_ref[4] + ab_ref[4])[:, 0:16]
        rcp = 1.0 / jnp.maximum(at[:, 3:4], 1.0)
        r16 = _mm(rcp, jnp.ones((16, 1), F32))
        co_ref[...] = cp_ref[...] + at * r16

    full = lambda shape: pl.BlockSpec(shape, lambda i: (0,) * len(shape))
    return pl.pallas_call(
        body,
        grid=(N // TN,),
        in_specs=[
            pl.BlockSpec((TN, HID), lambda i: (i, 0)),
            pl.BlockSpec((5, TN, 128), lambda i: (0, i, 0)),
            pl.BlockSpec((5, TN, 128), lambda i: (0, i, 0)),
            pl.BlockSpec((TN, 16), lambda i: (i, 0)),
            pl.BlockSpec((TN, IN_SIZE), lambda i: (i, 0)),
            full((HID, HID)),
            full((1, HID)),
            full((HID, HID)),
            full((1, HID)),
            full((IN_SIZE, HID)),
            full((1, IN_SIZE)),
        ],
        out_specs=[
            pl.BlockSpec((TN, IN_SIZE), lambda i: (i, 0)),
            pl.BlockSpec((TN, 16), lambda i: (i, 0)),
        ],
        out_shape=[
            jax.ShapeDtypeStruct((N, IN_SIZE), F32),
            jax.ShapeDtypeStruct((N, 16), F32),
        ],
    )(hn, agg_a, agg_b, coordp, inp, wn1a, bn1, wn2, bn2, w_out, b_out)


def kernel(input, coord_feat, h0, lamda, alpha, l, edge_index, efeats,
           W_emb_in, b_emb_in, We1, be1, We2, be2, Wn1, bn1, Wn2, bn2,
           Wc1, bc1, Wc2, Wa, ba, W_emb_out, b_emb_out):
    row = edge_index[0].astype(jnp.int32)
    col = edge_index[1].astype(jnp.int32)
    coordp = jnp.pad(coord_feat, ((0, 0), (0, 13)))

    we1r = We1[:, :HID]
    we1c = We1[:, HID:2 * HID]
    wrad8 = jnp.tile(We1[:, 2 * HID].reshape(HID, 1), (1, 8))
    we1ef = We1[:, 2 * HID + 1:]
    wn1h = Wn1[:, :HID]
    wn1a = Wn1[:, HID:]
    wa_rep = jnp.tile(Wa, (128, 1))
    ba128 = jnp.tile(ba.reshape(1, 1), (1, 128))
    wc2_rep = jnp.tile(Wc2, (128, 1))
    r2 = lambda v: v.reshape(1, -1)

    hse, hce, hn = _tc_precompute(input, coordp, W_emb_in, r2(b_emb_in),
                                  we1r, we1c, wn1h)

    edge_args = (wrad8, we1ef, r2(be1), We2, r2(be2), wa_rep, ba128,
                 Wc1, r2(bc1), wc2_rep)
    g1a, g2a = _sc_gather(hse, hce, row[:EH], col[:EH])
    g1b, g2b = _sc_gather(hse, hce, row[EH:], col[EH:])
    m5a = _tc_edge(g1a, g2a, efeats[:EH], *edge_args)
    agg_a = _sc_scatter(m5a, row[:EH], 0)
    m5b = _tc_edge(g1b, g2b, efeats[EH:], *edge_args)
    agg_b = _sc_scatter(m5b, row[EH:], 1)

    out, co16 = _tc_node(hn, agg_a, agg_b, coordp, input, wn1a,
                         r2(bn1), Wn2, r2(bn2), W_emb_out,
                         r2(b_emb_out))
    return out, co16[:, 0:3]


# speedup vs baseline: 3.6961x; 1.0420x over previous
import functools

import jax
import jax.numpy as jnp
from jax import lax
from jax.experimental import pallas as pl
from jax.experimental.pallas import tpu as pltpu
from jax.experimental.pallas import tpu_sc as plsc

N = 10000
NP = 10240
E = 160000
EH = E // 2
IN_SIZE = 256
HID = 512
EDGE_NF = 16
GW = HID + 128

NC = 2
NS = 16
NW = NC * NS

F32 = jnp.float32


def _mm(x, w):
    return lax.dot_general(x, w, (((1,), (1,)), ((), ())),
                           preferred_element_type=F32)


def _tc_precompute(inp, coordp, w_emb, b_emb, we1r, we1c, wn1h):
    TN = 2000

    def body(x_ref, cp_ref, wemb_ref, bemb_ref, wr_ref, wc_ref, wn_ref,
             hs_ref, hc_ref, hn_ref):
        h = _mm(x_ref[...], wemb_ref[...]) + bemb_ref[...]
        z = jnp.zeros((h.shape[0], GW - HID - 16), F32)
        cp = cp_ref[...]
        hs_ref[...] = jnp.concatenate([_mm(h, wr_ref[...]), cp, z], axis=1)
        hc_ref[...] = jnp.concatenate([_mm(h, wc_ref[...]), cp, z], axis=1)
        hn_ref[...] = _mm(h, wn_ref[...])

    full = lambda shape: pl.BlockSpec(shape, lambda i: (0,) * len(shape))
    return pl.pallas_call(
        body,
        grid=(N // TN,),
        in_specs=[
            pl.BlockSpec((TN, IN_SIZE), lambda i: (i, 0)),
            pl.BlockSpec((TN, 16), lambda i: (i, 0)),
            full((HID, IN_SIZE)),
            full((1, HID)),
            full((HID, HID)),
            full((HID, HID)),
            full((HID, HID)),
        ],
        out_specs=[
            pl.BlockSpec((TN, GW), lambda i: (i, 0)),
            pl.BlockSpec((TN, GW), lambda i: (i, 0)),
            pl.BlockSpec((TN, HID), lambda i: (i, 0)),
        ],
        out_shape=[
            jax.ShapeDtypeStruct((N, GW), F32),
            jax.ShapeDtypeStruct((N, GW), F32),
            jax.ShapeDtypeStruct((N, HID), F32),
        ],
    )(inp, coordp, w_emb, b_emb, we1r, we1c, wn1h)


def _sc_gather(hse, hce, row, col):
    EPS = EH // NS
    B = 40
    NB = EPS // B
    NPAIR = NB // 2
    mesh = plsc.VectorSubcoreMesh(core_axis_name="c", subcore_axis_name="s")

    @functools.partial(
        pl.kernel,
        mesh=mesh,
        out_type=(
            jax.ShapeDtypeStruct((EH, GW), F32),
            jax.ShapeDtypeStruct((EH, GW), F32),
        ),
        scratch_types=[
            pltpu.VMEM((B,), jnp.int32),
            pltpu.VMEM((B,), jnp.int32),
            pltpu.VMEM((B, GW), F32),
            pltpu.VMEM((B, GW), F32),
            pltpu.SemaphoreType.DMA,
            pltpu.SemaphoreType.DMA,
            pltpu.SemaphoreType.DMA,
            pltpu.SemaphoreType.DMA,
        ],
    )
    def k(hs_h, hc_h, row_h, col_h, g1_h, g2_h,
          iv0, iv1, gb0, gb1, sg0, sg1, sw0, sw1):
        c = lax.axis_index("c")
        s = lax.axis_index("s")
        iv = [iv0, iv1]
        gb = [gb0, gb1]
        sg = [sg0, sg1]
        sw = [sw0, sw1]

        def one_side(src_h, idx_h, out_h):
            @pl.loop(0, NPAIR)
            def pair(bb):
                for p in range(2):
                    base = s * EPS + (bb * 2 + p) * B

                    @pl.when(bb >= 1)
                    def wait_write(base=base, p=p):
                        pltpu.make_async_copy(
                            gb[p], out_h.at[pl.ds(base - 2 * B, B)],
                            sw[p]).wait()

                    pltpu.sync_copy(idx_h.at[pl.ds(base, B)], iv[p])
                    pltpu.async_copy(src_h.at[iv[p]], gb[p], sg[p])
                for p in range(2):
                    base = s * EPS + (bb * 2 + p) * B
                    pltpu.make_async_copy(src_h.at[iv[p]], gb[p],
                                          sg[p]).wait()
                    pltpu.async_copy(gb[p], out_h.at[pl.ds(base, B)], sw[p])

            for p in range(2):
                base = s * EPS + ((NPAIR - 1) * 2 + p) * B
                pltpu.make_async_copy(gb[p], out_h.at[pl.ds(base, B)],
                                      sw[p]).wait()
            base = s * EPS + (NB - 1) * B
            pltpu.sync_copy(idx_h.at[pl.ds(base, B)], iv[0])
            pltpu.async_copy(src_h.at[iv[0]], gb[0], sg[0])
            pltpu.make_async_copy(src_h.at[iv[0]], gb[0], sg[0]).wait()
            pltpu.sync_copy(gb[0], out_h.at[pl.ds(base, B)])

        @pl.when(c == 0)
        def side1():
            one_side(hs_h, row_h, g1_h)

        @pl.when(c == 1)
        def side2():
            one_side(hc_h, col_h, g2_h)

    return k(hse, hce, row, col)


def _tc_edge(g1, g2, efeats, wrad8, we1ef, be1, we2, be2,
             wa_rep, ba128, wc1, bc1, wc2_rep):
    TE = 1600

    def body(g1h_ref, g1c_ref, g2h_ref, g2c_ref, ef_ref,
             wrad8_ref, we1ef_ref, be1_ref, we2_ref, be2_ref,
             wa_ref, ba_ref, wc1_ref, bc1_ref, wc2_ref,
             m5_ref):
        d = g1c_ref[...] - g2c_ref[...]
        d8 = d[:, 0:8]
        pre = (g1h_ref[...] + g2h_ref[...] + _mm(d8 * d8, wrad8_ref[...])
               + _mm(ef_ref[...], we1ef_ref[...]) + be1_ref[...])
        m1 = jax.nn.silu(pre)
        m2 = jax.nn.silu(_mm(m1, we2_ref[...]) + be2_ref[...])
        att = jax.nn.sigmoid(_mm(m2, wa_ref[...]) + ba_ref[...])
        attc = jnp.concatenate([att] * 4, axis=1)
        m = m2 * attc
        for ci in range(4):
            m5_ref[ci] = m[:, 128 * ci:128 * (ci + 1)]
        cfeat = jax.nn.silu(_mm(m, wc1_ref[...]) + bc1_ref[...])
        s = _mm(cfeat, wc2_ref[...])
        lane = lax.broadcasted_iota(jnp.int32, d.shape, 1)
        m5_ref[4] = d * s + jnp.where(lane == 3, 1.0, 0.0)

    full = lambda shape: pl.BlockSpec(shape, lambda i: (0,) * len(shape))
    return pl.pallas_call(
        body,
        grid=(EH // TE,),
        in_specs=[
            pl.BlockSpec((TE, HID), lambda i: (i, 0)),
            pl.BlockSpec((TE, 128), lambda i: (i, 4)),
            pl.BlockSpec((TE, HID), lambda i: (i, 0)),
            pl.BlockSpec((TE, 128), lambda i: (i, 4)),
            pl.BlockSpec((TE, EDGE_NF), lambda i: (i, 0)),
            full((HID, 8)),
            full((HID, EDGE_NF)),
            full((1, HID)),
            full((HID, HID)),
            full((1, HID)),
            full((128, HID)),
            full((1, 128)),
            full((HID, HID)),
            full((1, HID)),
            full((128, HID)),
        ],
        out_specs=[pl.BlockSpec((5, TE, 128), lambda i: (0, i, 0))],
        out_shape=[jax.ShapeDtypeStruct((5, EH, 128), F32)],
    )(g1, g1, g2, g2, efeats, wrad8, we1ef, be1, we2, be2, wa_rep, ba128,
      wc1, bc1, wc2_rep)[0]


def _sc_scatter(m5, row):
    EPS = EH // NS
    BS = 40
    NZ = 128
    RPT = NP // NS
    mesh = plsc.VectorSubcoreMesh(core_axis_name="c", subcore_axis_name="s")

    @functools.partial(
        pl.kernel,
        mesh=mesh,
        out_type=jax.ShapeDtypeStruct((6, NP, 128), F32),
        scratch_types=[
            pltpu.VMEM((BS,), jnp.int32),
            pltpu.VMEM((BS,), jnp.int32),
            pltpu.VMEM((BS, 128), F32),
            pltpu.VMEM((BS, 128), F32),
            pltpu.VMEM((NZ, 128), F32),
            pltpu.VMEM_SHARED((NP, 128), F32),
            pltpu.SemaphoreType.DMA,
            pltpu.SemaphoreType.DMA,
            pltpu.SemaphoreType.DMA,
            pltpu.SemaphoreType.DMA,
        ],
    )
    def k(m5_h, row_h, agg_h, ib0, ib1, mb0, mb1,
          zb, acc, si0, si1, sm0, sm1):
        c = lax.axis_index("c")
        s = lax.axis_index("s")
        ib = [ib0, ib1]
        mb = [mb0, mb1]
        si = [si0, si1]
        sm = [sm0, sm1]

        @pl.loop(0, NZ)
        def zr(r):
            @pl.loop(0, 8)
            def zc(j):
                zb[r, pl.ds(j * 16, 16)] = jnp.zeros((16,), F32)

        def zero_acc():
            @pl.loop(0, RPT // NZ)
            def zz(z):
                pltpu.sync_copy(zb, acc.at[pl.ds(s * RPT + z * NZ, NZ)])

        def writeout(plane):
            @pl.loop(0, RPT // NZ)
            def wz(z):
                r0 = s * RPT + z * NZ
                pltpu.sync_copy(acc.at[pl.ds(r0, NZ)],
                                agg_h.at[plane, pl.ds(r0, NZ)])

        def sweep(ch, mult, offs, nbk):
            first = s * EPS
            npair = (nbk - 1) // 2

            def start(kk, p):
                base = first + (mult * kk + offs) * BS
                pltpu.async_copy(row_h.at[pl.ds(base, BS)], ib[p], si[p])
                pltpu.async_copy(m5_h.at[ch, pl.ds(base, BS)], mb[p], sm[p])

            def flush(p):
                pltpu.make_async_copy(row_h.at[pl.ds(first, BS)],
                                      ib[p], si[p]).wait()
                pltpu.make_async_copy(m5_h.at[ch, pl.ds(first, BS)],
                                      mb[p], sm[p]).wait()
                pltpu.sync_copy(mb[p], acc.at[ib[p]], add=True)

            start(0, 0)

            @pl.loop(0, npair)
            def pr(bb):
                for (p, off) in ((1, 1), (0, 2)):
                    start(2 * bb + off, p)
                    flush(1 - p)

            flush(0)
            if nbk % 2 == 0:
                start(nbk - 1, 1)
                flush(1)

        NB = EPS // BS

        for ch_i in range(2):
            zero_acc()
            plsc.subcore_barrier()
            sweep(c * 2 + ch_i, 1, 0, NB)
            plsc.subcore_barrier()
            writeout(c * 2 + ch_i)
            plsc.subcore_barrier()

        zero_acc()
        plsc.subcore_barrier()

        @pl.when(c == 0)
        def c4_sweep0():
            sweep(4, 2, 0, (NB + 1) // 2)

        @pl.when(c == 1)
        def c4_sweep1():
            sweep(4, 2, 1, NB // 2)

        plsc.subcore_barrier()
        writeout(4 + c)
        plsc.subcore_barrier()

    return k(m5, row)


def _tc_node(hn, agg_a, agg_b, coordp, inp, wn1a, bn1, wn2, bn2,
             w_out, b_out):
    TN = 2000

    def body(hn_ref, aa_ref, ab_ref, cp_ref, x_ref,
             wn1a_ref, bn1_ref, wn2_ref, bn2_ref, wo_ref, bo_ref,
             out_ref, co_ref):
        wn1a = wn1a_ref[...]
        pre = hn_ref[...] + bn1_ref[...]
        for ci in range(4):
            agg = aa_ref[ci] + ab_ref[ci]
            pre = pre + _mm(agg, wn1a[:, 128 * ci:128 * (ci + 1)])
        nh = jax.nn.silu(pre)
        h2 = _mm(nh, wn2_ref[...]) + bn2_ref[...]
        out_ref[...] = _mm(h2, wo_ref[...]) + bo_ref[...] + x_ref[...]
        at = (aa_ref[4] + aa_ref[5]
              + ab_ref[4] + ab_ref[5])[:, 0:16]
        rcp = 1.0 / jnp.maximum(at[:, 3:4], 1.0)
        r16 = _mm(rcp, jnp.ones((16, 1), F32))
        co_ref[...] = cp_ref[...] + at * r16

    full = lambda shape: pl.BlockSpec(shape, lambda i: (0,) * len(shape))
    return pl.pallas_call(
        body,
        grid=(N // TN,),
        in_specs=[
            pl.BlockSpec((TN, HID), lambda i: (i, 0)),
            pl.BlockSpec((6, TN, 128), lambda i: (0, i, 0)),
            pl.BlockSpec((6, TN, 128), lambda i: (0, i, 0)),
            pl.BlockSpec((TN, 16), lambda i: (i, 0)),
            pl.BlockSpec((TN, IN_SIZE), lambda i: (i, 0)),
            full((HID, HID)),
            full((1, HID)),
            full((HID, HID)),
            full((1, HID)),
            full((IN_SIZE, HID)),
            full((1, IN_SIZE)),
        ],
        out_specs=[
            pl.BlockSpec((TN, IN_SIZE), lambda i: (i, 0)),
            pl.BlockSpec((TN, 16), lambda i: (i, 0)),
        ],
        out_shape=[
            jax.ShapeDtypeStruct((N, IN_SIZE), F32),
            jax.ShapeDtypeStruct((N, 16), F32),
        ],
    )(hn, agg_a, agg_b, coordp, inp, wn1a, bn1, wn2, bn2, w_out, b_out)


def kernel(input, coord_feat, h0, lamda, alpha, l, edge_index, efeats,
           W_emb_in, b_emb_in, We1, be1, We2, be2, Wn1, bn1, Wn2, bn2,
           Wc1, bc1, Wc2, Wa, ba, W_emb_out, b_emb_out):
    row = edge_index[0].astype(jnp.int32)
    col = edge_index[1].astype(jnp.int32)
    coordp = jnp.pad(coord_feat, ((0, 0), (0, 13)))

    we1r = We1[:, :HID]
    we1c = We1[:, HID:2 * HID]
    wrad8 = jnp.tile(We1[:, 2 * HID].reshape(HID, 1), (1, 8))
    we1ef = We1[:, 2 * HID + 1:]
    wn1h = Wn1[:, :HID]
    wn1a = Wn1[:, HID:]
    wa_rep = jnp.tile(Wa, (128, 1))
    ba128 = jnp.tile(ba.reshape(1, 1), (1, 128))
    wc2_rep = jnp.tile(Wc2, (128, 1))
    r2 = lambda v: v.reshape(1, -1)

    hse, hce, hn = _tc_precompute(input, coordp, W_emb_in, r2(b_emb_in),
                                  we1r, we1c, wn1h)

    edge_args = (wrad8, we1ef, r2(be1), We2, r2(be2), wa_rep, ba128,
                 Wc1, r2(bc1), wc2_rep)
    g1a, g2a = _sc_gather(hse, hce, row[:EH], col[:EH])
    g1b, g2b = _sc_gather(hse, hce, row[EH:], col[EH:])
    m5a = _tc_edge(g1a, g2a, efeats[:EH], *edge_args)
    agg_a = _sc_scatter(m5a, row[:EH])
    m5b = _tc_edge(g1b, g2b, efeats[EH:], *edge_args)
    agg_b = _sc_scatter(m5b, row[EH:])

    out, co16 = _tc_node(hn, agg_a, agg_b, coordp, input, wn1a,
                         r2(bn1), Wn2, r2(bn2), W_emb_out,
                         r2(b_emb_out))
    return out, co16[:, 0:3]
